# Initial kernel scaffold; baseline (speedup 1.0000x reference)
import jax
import jax.numpy as jnp
from jax.experimental import pallas as pl


def kernel(x, Wg, W1, b1, W2, b2):
    raise NotImplementedError("write your pallas kernel here")


# speedup vs baseline: 1.5007x; 1.5007x over previous
import functools

import jax
import jax.numpy as jnp
from jax.experimental import pallas as pl
from jax.experimental.pallas import tpu as pltpu

_E = 8
_C = 2
_FT = 512


def _moe_body(x_ref, w1_ref, b1_ref, w2_ref, b2_ref, g_ref, i_ref, out_ref,
              xe_ref, yacc_ref, *, T, K, F):
    e = pl.program_id(0)
    f = pl.program_id(1)

    @pl.when(f == 0)
    def _gather():
        idx = i_ref[0, 0, :]
        tok = jax.lax.broadcasted_iota(jnp.int32, (K, T), 1)
        p = (idx[:, None] == tok).astype(jnp.bfloat16)
        xe = jnp.dot(p, x_ref[...].astype(jnp.bfloat16),
                     preferred_element_type=jnp.float32)
        xe_ref[...] = xe.astype(jnp.bfloat16)

    h = jnp.dot(xe_ref[...], w1_ref[0].astype(jnp.bfloat16),
                preferred_element_type=jnp.float32)
    h = jax.nn.gelu(h + b1_ref[0, 0][None, :], approximate=True)
    y = jnp.dot(h.astype(jnp.bfloat16), w2_ref[0].astype(jnp.bfloat16),
                preferred_element_type=jnp.float32)

    @pl.when(f == 0)
    def _init_yacc():
        yacc_ref[...] = jnp.zeros_like(yacc_ref)

    yacc_ref[...] += y

    @pl.when(jnp.logical_and(e == 0, f == 0))
    def _init_out():
        out_ref[...] = jnp.zeros_like(out_ref)

    @pl.when(f == F - 1)
    def _scatter():
        idx = i_ref[0, 0, :]
        yk = g_ref[0, 0, :][:, None] * (yacc_ref[...] + b2_ref[0, 0][None, :])
        tok = jax.lax.broadcasted_iota(jnp.int32, (T, K), 0)
        pt = (tok == idx[None, :]).astype(jnp.bfloat16)
        out_ref[...] += jnp.dot(pt, yk.astype(jnp.bfloat16),
                                preferred_element_type=jnp.float32)


def _moe_tc(xf, W1, b1, W2, b2, G, I, *, interpret=False):
    T, D = xf.shape
    E, _, DFF = W1.shape
    K = G.shape[-1]
    F = DFF // _FT
    grid = (E, F)
    body = functools.partial(_moe_body, T=T, K=K, F=F)
    return pl.pallas_call(
        body,
        grid=grid,
        in_specs=[
            pl.BlockSpec((T, D), lambda e, f: (0, 0)),
            pl.BlockSpec((1, D, _FT), lambda e, f: (e, 0, f)),
            pl.BlockSpec((1, 1, _FT), lambda e, f: (e * F + f, 0, 0)),
            pl.BlockSpec((1, _FT, D), lambda e, f: (e, f, 0)),
            pl.BlockSpec((1, 1, D), lambda e, f: (e, 0, 0)),
            pl.BlockSpec((1, 1, K), lambda e, f: (e, 0, 0)),
            pl.BlockSpec((1, 1, K), lambda e, f: (e, 0, 0)),
        ],
        out_specs=pl.BlockSpec((T, D), lambda e, f: (0, 0)),
        out_shape=jax.ShapeDtypeStruct((T, D), jnp.float32),
        scratch_shapes=[
            pltpu.VMEM((K, D), jnp.bfloat16),
            pltpu.VMEM((K, D), jnp.float32),
        ],
        compiler_params=pltpu.CompilerParams(
            dimension_semantics=("arbitrary", "arbitrary"),
        ),
        interpret=interpret,
    )(xf, W1, b1.reshape(E * F, 1, _FT), W2, b2.reshape(E, 1, D), G, I)


def kernel(x, Wg, W1, b1, W2, b2):
    b, l, d = x.shape
    xf = x.reshape(b * l, d)
    T = b * l
    E = W1.shape[0]
    k = min(max(int(T * _C / E), 1), T)
    S = jax.nn.sigmoid(xf @ Wg)
    G, I = jax.lax.top_k(S.T, k)
    out = _moe_tc(xf, W1, b1, W2, b2,
                  G.reshape(E, 1, k), I.astype(jnp.int32).reshape(E, 1, k))
    return out.reshape(b, l, d)
